# baseline (speedup 1.0000x reference)
import functools

import jax
import jax.numpy as jnp
from jax import lax
from jax.experimental import pallas as pl
from jax.experimental.pallas import tpu as pltpu
from jax.experimental.pallas import tpu_sc as plsc

_NC = 2
_NS = 16
_B = 40


def _tables_body(h_ref, wms_ref, wmr_ref, ts_ref, tr_ref):
    h = h_ref[...]
    ts_ref[...] = jnp.dot(h, wms_ref[...], preferred_element_type=jnp.float32)
    tr_ref[...] = jnp.dot(h, wmr_ref[...], preferred_element_type=jnp.float32)


def _ew_body(e_ref, wme_ref, b_ref, out_ref):
    out_ref[...] = (
        jnp.dot(e_ref[...], wme_ref[...], preferred_element_type=jnp.float32)
        + b_ref[...]
    )


def _enew_packed_body(e8_ref, m8_ref, wee_ref, wem_ref, b_ref, out_ref):
    acc = jnp.dot(e8_ref[...], wee_ref[...], preferred_element_type=jnp.float32)
    acc += jnp.dot(m8_ref[...], wem_ref[...], preferred_element_type=jnp.float32)
    out_ref[...] = jnp.maximum(acc + b_ref[...], 0.0)


def _hnew_body(h_ref, a0_ref, a1_ref, wnh_ref, wna_ref, b_ref, out_ref):
    acc = jnp.dot(h_ref[...], wnh_ref[...], preferred_element_type=jnp.float32)
    acc += jnp.dot(a0_ref[0] + a1_ref[0], wna_ref[...],
                   preferred_element_type=jnp.float32)
    out_ref[...] = jnp.maximum(acc + b_ref[...], 0.0)


def _enew_body(e_ref, m_ref, wee_ref, wem_ref, b_ref, out_ref):
    acc = jnp.dot(e_ref[...], wee_ref[...], preferred_element_type=jnp.float32)
    acc += jnp.dot(m_ref[...], wem_ref[...], preferred_element_type=jnp.float32)
    out_ref[...] = jnp.maximum(acc + b_ref[...], 0.0)


@functools.lru_cache(maxsize=None)
def _make_sc_messages(E, N_pad, F):
    per_tile = E // (_NC * _NS)
    assert per_tile * _NC * _NS == E
    n_chunks = per_tile // _B
    assert n_chunks * _B == per_tile and n_chunks % 2 == 0
    rows_per_sub = N_pad // _NS
    assert rows_per_sub * _NS == N_pad and rows_per_sub % 8 == 0

    mesh = plsc.VectorSubcoreMesh(core_axis_name="c", subcore_axis_name="s",
                                  num_cores=_NC, num_subcores=_NS)

    @functools.partial(
        pl.kernel,
        out_type=[
            jax.ShapeDtypeStruct((E, F), jnp.float32),
            jax.ShapeDtypeStruct((_NC, N_pad, F), jnp.float32),
        ],
        mesh=mesh,
        scratch_types=[
            pltpu.VMEM((_B,), jnp.int32),
            pltpu.VMEM((_B,), jnp.int32),
            pltpu.VMEM((_B,), jnp.int32),
            pltpu.VMEM((_B,), jnp.int32),
            pltpu.VMEM((_B, F), jnp.float32),
            pltpu.VMEM((_B, F), jnp.float32),
            pltpu.VMEM((_B, F), jnp.float32),
            pltpu.VMEM((_B, F), jnp.float32),
            pltpu.VMEM((_B, F), jnp.float32),
            pltpu.VMEM((_B, F), jnp.float32),
            pltpu.VMEM_SHARED((N_pad, F), jnp.float32),
            pltpu.SemaphoreType.DMA,
            pltpu.SemaphoreType.DMA,
            pltpu.SemaphoreType.DMA,
            pltpu.SemaphoreType.DMA,
            pltpu.SemaphoreType.DMA,
            pltpu.SemaphoreType.DMA,
        ],
    )
    def sc_messages(ew, snd, rcv, ts, tr, zeros, m_out, agg_out,
                    idx_s0, idx_s1, idx_r0, idx_r1,
                    buf_ew0, buf_ew1, buf_s0, buf_s1, buf_r0, buf_r1,
                    agg_sh, isem0, isem1, gsem0, gsem1, msem0, msem1):
        idx_s = (idx_s0, idx_s1)
        idx_r = (idx_r0, idx_r1)
        buf_ew = (buf_ew0, buf_ew1)
        buf_s = (buf_s0, buf_s1)
        buf_r = (buf_r0, buf_r1)
        isem = (isem0, isem1)
        gsem = (gsem0, gsem1)
        msem = (msem0, msem1)
        cid = lax.axis_index("c")
        sid = lax.axis_index("s")
        wid = cid * _NS + sid
        base0 = wid * per_tile

        r0 = sid * rows_per_sub
        pltpu.sync_copy(zeros.at[pl.ds(r0, rows_per_sub)],
                        agg_sh.at[pl.ds(r0, rows_per_sub)])
        plsc.subcore_barrier()

        def issue_idx(c, slot):
            base = base0 + c * _B
            pltpu.async_copy(snd.at[pl.ds(base, _B)], idx_s[slot], isem[slot])
            pltpu.async_copy(rcv.at[pl.ds(base, _B)], idx_r[slot], isem[slot])

        def wait_idx(slot):
            pltpu.make_async_copy(snd.at[pl.ds(0, _B)], idx_s[slot],
                                  isem[slot]).wait()
            pltpu.make_async_copy(rcv.at[pl.ds(0, _B)], idx_r[slot],
                                  isem[slot]).wait()

        def issue_data(c, slot):
            pltpu.async_copy(ew.at[pl.ds(base0 + c * _B, _B)], buf_ew[slot],
                             gsem[slot])
            pltpu.async_copy(ts.at[idx_s[slot]], buf_s[slot], gsem[slot])
            pltpu.async_copy(tr.at[idx_r[slot]], buf_r[slot], gsem[slot])

        def wait_data(slot):
            pltpu.make_async_copy(ew.at[pl.ds(0, _B)], buf_ew[slot],
                                  gsem[slot]).wait()
            pltpu.make_async_copy(ts.at[pl.ds(0, _B)], buf_s[slot],
                                  gsem[slot]).wait()
            pltpu.make_async_copy(tr.at[pl.ds(0, _B)], buf_r[slot],
                                  gsem[slot]).wait()

        def step(g, k, o):
            @pl.when(g >= 1)
            def _():
                pltpu.make_async_copy(buf_ew[o], m_out.at[pl.ds(0, _B)],
                                      msem[o]).wait()

            @pl.when(g + 1 < n_chunks)
            def _():
                wait_idx(o)
                issue_data(g + 1, o)

            wait_data(k)

            def row_body(i, rc):
                for j in range(F // 16):
                    sl = pl.ds(j * 16, 16)
                    v = buf_ew[k][i, sl] + buf_s[k][i, sl] + buf_r[k][i, sl]
                    buf_ew[k][i, sl] = jnp.maximum(v, 0.0)
                return rc

            lax.fori_loop(0, _B, row_body, 0)
            pltpu.async_copy(buf_ew[k], m_out.at[pl.ds(base0 + g * _B, _B)],
                             msem[k])
            pltpu.sync_copy(buf_ew[k], agg_sh.at[idx_r[k]], add=True)

            @pl.when(g + 2 < n_chunks)
            def _():
                issue_idx(g + 2, k)

        issue_idx(0, 0)
        issue_idx(1, 1)
        wait_idx(0)
        issue_data(0, 0)

        def pair_body(t, carry):
            g0 = 2 * t
            step(g0, 0, 1)
            step(g0 + 1, 1, 0)
            return carry

        lax.fori_loop(0, n_chunks // 2, pair_body, 0)
        last_slot = (n_chunks - 1) % 2
        pltpu.make_async_copy(buf_ew[last_slot], m_out.at[pl.ds(0, _B)],
                              msem[last_slot]).wait()

        plsc.subcore_barrier()
        pltpu.sync_copy(agg_sh.at[pl.ds(r0, rows_per_sub)],
                        agg_out.at[cid, pl.ds(r0, rows_per_sub)])

    return sc_messages


def kernel(h, e, senders, receivers, W_msg, b_msg, W_node, b_node, W_edge, b_edge):
    N, F = h.shape
    E, De = e.shape
    senders = senders.astype(jnp.int32)
    receivers = receivers.astype(jnp.int32)

    wm_e = W_msg[:De]
    wm_s = W_msg[De:De + F]
    wm_r = W_msg[De + F:]
    wn_h = W_node[:F]
    wn_a = W_node[F:]
    we_e = W_edge[:De]
    we_m = W_edge[De:]
    b_msg2 = b_msg.reshape(1, F)
    b_node2 = b_node.reshape(1, F)

    pk = F // De
    e8 = e.reshape(E // pk, F)
    wm_e_blk = jax.scipy.linalg.block_diag(*([wm_e] * pk))
    we_e_blk = jax.scipy.linalg.block_diag(*([we_e] * pk))
    we_m_blk = jax.scipy.linalg.block_diag(*([we_m] * pk))
    b_msg8 = jnp.tile(b_msg, pk).reshape(1, pk * F)
    b_edge8 = jnp.tile(b_edge, pk).reshape(1, F)

    bn = 1000
    be = 8000
    bp = be // pk

    ts, tr = pl.pallas_call(
        _tables_body,
        grid=(N // bn,),
        in_specs=[
            pl.BlockSpec((bn, F), lambda i: (i, 0)),
            pl.BlockSpec((F, F), lambda i: (0, 0)),
            pl.BlockSpec((F, F), lambda i: (0, 0)),
        ],
        out_specs=[
            pl.BlockSpec((bn, F), lambda i: (i, 0)),
            pl.BlockSpec((bn, F), lambda i: (i, 0)),
        ],
        out_shape=[
            jax.ShapeDtypeStruct((N, F), jnp.float32),
            jax.ShapeDtypeStruct((N, F), jnp.float32),
        ],
    )(h, wm_s, wm_r)

    ew8 = pl.pallas_call(
        _ew_body,
        grid=(E // be,),
        in_specs=[
            pl.BlockSpec((bp, F), lambda i: (i, 0)),
            pl.BlockSpec((F, pk * F), lambda i: (0, 0)),
            pl.BlockSpec((1, pk * F), lambda i: (0, 0)),
        ],
        out_specs=pl.BlockSpec((bp, pk * F), lambda i: (i, 0)),
        out_shape=jax.ShapeDtypeStruct((E // pk, pk * F), jnp.float32),
    )(e8, wm_e_blk, b_msg8)
    ew = ew8.reshape(E, F)

    n_pad = ((N + (8 * _NS) - 1) // (8 * _NS)) * (8 * _NS)
    zeros = jnp.zeros((n_pad, F), jnp.float32)
    m, agg_parts = _make_sc_messages(E, n_pad, F)(
        ew, senders, receivers, ts, tr, zeros)

    h_new = pl.pallas_call(
        _hnew_body,
        grid=(N // bn,),
        in_specs=[
            pl.BlockSpec((bn, F), lambda i: (i, 0)),
            pl.BlockSpec((1, bn, F), lambda i: (0, i, 0)),
            pl.BlockSpec((1, bn, F), lambda i: (1, i, 0)),
            pl.BlockSpec((F, F), lambda i: (0, 0)),
            pl.BlockSpec((F, F), lambda i: (0, 0)),
            pl.BlockSpec((1, F), lambda i: (0, 0)),
        ],
        out_specs=pl.BlockSpec((bn, F), lambda i: (i, 0)),
        out_shape=jax.ShapeDtypeStruct((N, F), jnp.float32),
    )(h, agg_parts, agg_parts, wn_h, wn_a, b_node2)

    m8 = m.reshape(E // pk, pk * F)
    e_new8 = pl.pallas_call(
        _enew_packed_body,
        grid=(E // be,),
        in_specs=[
            pl.BlockSpec((bp, F), lambda i: (i, 0)),
            pl.BlockSpec((bp, pk * F), lambda i: (i, 0)),
            pl.BlockSpec((F, F), lambda i: (0, 0)),
            pl.BlockSpec((pk * F, F), lambda i: (0, 0)),
            pl.BlockSpec((1, F), lambda i: (0, 0)),
        ],
        out_specs=pl.BlockSpec((bp, F), lambda i: (i, 0)),
        out_shape=jax.ShapeDtypeStruct((E // pk, F), jnp.float32),
    )(e8, m8, we_e_blk, we_m_blk, b_edge8)
    e_new = e_new8.reshape(E, De)

    return h_new, e_new


# speedup vs baseline: 1.1970x; 1.1970x over previous
import functools

import jax
import jax.numpy as jnp
from jax import lax
from jax.experimental import pallas as pl
from jax.experimental.pallas import tpu as pltpu
from jax.experimental.pallas import tpu_sc as plsc

_NC = 2
_NS = 16
_B = 40


def _tables_body(h_ref, wms_ref, wmr_ref, ts_ref, tr_ref):
    h = h_ref[...]
    ts_ref[...] = jnp.dot(h, wms_ref[...], preferred_element_type=jnp.float32)
    tr_ref[...] = jnp.dot(h, wmr_ref[...], preferred_element_type=jnp.float32)


def _ew_packed_body(e8_ref, wblk_ref, b_ref, out_ref):
    acc = (
        jnp.dot(e8_ref[...], wblk_ref[...], preferred_element_type=jnp.float32)
        + b_ref[...]
    )
    for j in range(8):
        out_ref[:, j, :] = acc[:, 128 * j:128 * (j + 1)]


def _hnew_body(h_ref, a0_ref, a1_ref, wnh_ref, wna_ref, b_ref, out_ref):
    acc = jnp.dot(h_ref[...], wnh_ref[...], preferred_element_type=jnp.float32)
    acc += jnp.dot(a0_ref[0] + a1_ref[0], wna_ref[...],
                   preferred_element_type=jnp.float32)
    out_ref[...] = jnp.maximum(acc + b_ref[...], 0.0)


def _enew_body(e_ref, m_ref, wee_ref, wem_ref, b_ref, out_ref):
    acc = jnp.dot(e_ref[...], wee_ref[...], preferred_element_type=jnp.float32)
    acc += jnp.dot(m_ref[...], wem_ref[...], preferred_element_type=jnp.float32)
    out_ref[...] = jnp.maximum(acc + b_ref[...], 0.0)


@functools.lru_cache(maxsize=None)
def _make_sc_messages(E, N_pad, F):
    per_tile = E // (_NC * _NS)
    assert per_tile * _NC * _NS == E
    n_chunks = per_tile // _B
    assert n_chunks * _B == per_tile and n_chunks % 2 == 0
    rows_per_sub = N_pad // _NS
    assert rows_per_sub * _NS == N_pad and rows_per_sub % 8 == 0

    mesh = plsc.VectorSubcoreMesh(core_axis_name="c", subcore_axis_name="s",
                                  num_cores=_NC, num_subcores=_NS)

    @functools.partial(
        pl.kernel,
        out_type=[
            jax.ShapeDtypeStruct((E, F), jnp.float32),
            jax.ShapeDtypeStruct((_NC, N_pad, F), jnp.float32),
        ],
        mesh=mesh,
        scratch_types=[
            pltpu.VMEM((_B,), jnp.int32),
            pltpu.VMEM((_B,), jnp.int32),
            pltpu.VMEM((_B,), jnp.int32),
            pltpu.VMEM((_B,), jnp.int32),
            pltpu.VMEM((_B, F), jnp.float32),
            pltpu.VMEM((_B, F), jnp.float32),
            pltpu.VMEM((_B, F), jnp.float32),
            pltpu.VMEM((_B, F), jnp.float32),
            pltpu.VMEM((_B, F), jnp.float32),
            pltpu.VMEM((_B, F), jnp.float32),
            pltpu.VMEM_SHARED((N_pad, F), jnp.float32),
            pltpu.SemaphoreType.DMA,
            pltpu.SemaphoreType.DMA,
            pltpu.SemaphoreType.DMA,
            pltpu.SemaphoreType.DMA,
            pltpu.SemaphoreType.DMA,
            pltpu.SemaphoreType.DMA,
        ],
    )
    def sc_messages(ew, snd, rcv, ts, tr, zeros, m_out, agg_out,
                    idx_s0, idx_s1, idx_r0, idx_r1,
                    buf_ew0, buf_ew1, buf_s0, buf_s1, buf_r0, buf_r1,
                    agg_sh, isem0, isem1, gsem0, gsem1, msem0, msem1):
        idx_s = (idx_s0, idx_s1)
        idx_r = (idx_r0, idx_r1)
        buf_ew = (buf_ew0, buf_ew1)
        buf_s = (buf_s0, buf_s1)
        buf_r = (buf_r0, buf_r1)
        isem = (isem0, isem1)
        gsem = (gsem0, gsem1)
        msem = (msem0, msem1)
        cid = lax.axis_index("c")
        sid = lax.axis_index("s")
        wid = cid * _NS + sid
        base0 = wid * per_tile

        r0 = sid * rows_per_sub
        pltpu.sync_copy(zeros.at[pl.ds(r0, rows_per_sub)],
                        agg_sh.at[pl.ds(r0, rows_per_sub)])
        plsc.subcore_barrier()

        def issue_idx(c, slot):
            base = base0 + c * _B
            pltpu.async_copy(snd.at[pl.ds(base, _B)], idx_s[slot], isem[slot])
            pltpu.async_copy(rcv.at[pl.ds(base, _B)], idx_r[slot], isem[slot])

        def wait_idx(slot):
            pltpu.make_async_copy(snd.at[pl.ds(0, _B)], idx_s[slot],
                                  isem[slot]).wait()
            pltpu.make_async_copy(rcv.at[pl.ds(0, _B)], idx_r[slot],
                                  isem[slot]).wait()

        def issue_data(c, slot):
            pltpu.async_copy(ew.at[pl.ds(base0 + c * _B, _B)], buf_ew[slot],
                             gsem[slot])
            pltpu.async_copy(ts.at[idx_s[slot]], buf_s[slot], gsem[slot])
            pltpu.async_copy(tr.at[idx_r[slot]], buf_r[slot], gsem[slot])

        def wait_data(slot):
            pltpu.make_async_copy(ew.at[pl.ds(0, _B)], buf_ew[slot],
                                  gsem[slot]).wait()
            pltpu.make_async_copy(ts.at[pl.ds(0, _B)], buf_s[slot],
                                  gsem[slot]).wait()
            pltpu.make_async_copy(tr.at[pl.ds(0, _B)], buf_r[slot],
                                  gsem[slot]).wait()

        def step(g, k, o):
            @pl.when(g >= 1)
            def _():
                pltpu.make_async_copy(buf_ew[o], m_out.at[pl.ds(0, _B)],
                                      msem[o]).wait()

            @pl.when(g + 1 < n_chunks)
            def _():
                wait_idx(o)
                issue_data(g + 1, o)

            wait_data(k)

            def row_body(i, rc):
                for j in range(F // 16):
                    sl = pl.ds(j * 16, 16)
                    v = buf_ew[k][i, sl] + buf_s[k][i, sl] + buf_r[k][i, sl]
                    buf_ew[k][i, sl] = jnp.maximum(v, 0.0)
                return rc

            lax.fori_loop(0, _B, row_body, 0)
            pltpu.async_copy(buf_ew[k], m_out.at[pl.ds(base0 + g * _B, _B)],
                             msem[k])
            pltpu.sync_copy(buf_ew[k], agg_sh.at[idx_r[k]], add=True)

            @pl.when(g + 2 < n_chunks)
            def _():
                issue_idx(g + 2, k)

        issue_idx(0, 0)
        issue_idx(1, 1)
        wait_idx(0)
        issue_data(0, 0)

        def pair_body(t, carry):
            g0 = 2 * t
            step(g0, 0, 1)
            step(g0 + 1, 1, 0)
            return carry

        lax.fori_loop(0, n_chunks // 2, pair_body, 0)
        last_slot = (n_chunks - 1) % 2
        pltpu.make_async_copy(buf_ew[last_slot], m_out.at[pl.ds(0, _B)],
                              msem[last_slot]).wait()

        plsc.subcore_barrier()
        pltpu.sync_copy(agg_sh.at[pl.ds(r0, rows_per_sub)],
                        agg_out.at[cid, pl.ds(r0, rows_per_sub)])

    return sc_messages


def kernel(h, e, senders, receivers, W_msg, b_msg, W_node, b_node, W_edge, b_edge):
    N, F = h.shape
    E, De = e.shape
    senders = senders.astype(jnp.int32)
    receivers = receivers.astype(jnp.int32)

    wm_e = W_msg[:De]
    wm_s = W_msg[De:De + F]
    wm_r = W_msg[De + F:]
    wn_h = W_node[:F]
    wn_a = W_node[F:]
    we_e = W_edge[:De]
    we_m = W_edge[De:]
    b_msg2 = b_msg.reshape(1, F)
    b_node2 = b_node.reshape(1, F)

    pk = F // De
    e8 = e.reshape(E // pk, F)
    wm_e_blk = jax.scipy.linalg.block_diag(*([wm_e] * pk))
    b_msg8 = jnp.tile(b_msg, pk).reshape(1, pk * F)

    bn = 1000
    be = 8000
    bp = be // pk

    ts, tr = pl.pallas_call(
        _tables_body,
        grid=(N // bn,),
        in_specs=[
            pl.BlockSpec((bn, F), lambda i: (i, 0)),
            pl.BlockSpec((F, F), lambda i: (0, 0)),
            pl.BlockSpec((F, F), lambda i: (0, 0)),
        ],
        out_specs=[
            pl.BlockSpec((bn, F), lambda i: (i, 0)),
            pl.BlockSpec((bn, F), lambda i: (i, 0)),
        ],
        out_shape=[
            jax.ShapeDtypeStruct((N, F), jnp.float32),
            jax.ShapeDtypeStruct((N, F), jnp.float32),
        ],
    )(h, wm_s, wm_r)

    ew8 = pl.pallas_call(
        _ew_packed_body,
        grid=(E // be,),
        in_specs=[
            pl.BlockSpec((bp, F), lambda i: (i, 0)),
            pl.BlockSpec((F, pk * F), lambda i: (0, 0)),
            pl.BlockSpec((1, pk * F), lambda i: (0, 0)),
        ],
        out_specs=pl.BlockSpec((bp, pk, F), lambda i: (i, 0, 0)),
        out_shape=jax.ShapeDtypeStruct((E // pk, pk, F), jnp.float32),
    )(e8, wm_e_blk, b_msg8)
    ew = ew8.reshape(E, F)

    n_pad = ((N + (8 * _NS) - 1) // (8 * _NS)) * (8 * _NS)
    zeros = jnp.zeros((n_pad, F), jnp.float32)
    m, agg_parts = _make_sc_messages(E, n_pad, F)(
        ew, senders, receivers, ts, tr, zeros)

    h_new = pl.pallas_call(
        _hnew_body,
        grid=(N // bn,),
        in_specs=[
            pl.BlockSpec((bn, F), lambda i: (i, 0)),
            pl.BlockSpec((1, bn, F), lambda i: (0, i, 0)),
            pl.BlockSpec((1, bn, F), lambda i: (1, i, 0)),
            pl.BlockSpec((F, F), lambda i: (0, 0)),
            pl.BlockSpec((F, F), lambda i: (0, 0)),
            pl.BlockSpec((1, F), lambda i: (0, 0)),
        ],
        out_specs=pl.BlockSpec((bn, F), lambda i: (i, 0)),
        out_shape=jax.ShapeDtypeStruct((N, F), jnp.float32),
    )(h, agg_parts, agg_parts, wn_h, wn_a, b_node2)

    e_new = pl.pallas_call(
        _enew_body,
        grid=(E // be,),
        in_specs=[
            pl.BlockSpec((be, De), lambda i: (i, 0)),
            pl.BlockSpec((be, F), lambda i: (i, 0)),
            pl.BlockSpec((De, De), lambda i: (0, 0)),
            pl.BlockSpec((F, De), lambda i: (0, 0)),
            pl.BlockSpec((1, De), lambda i: (0, 0)),
        ],
        out_specs=pl.BlockSpec((be, De), lambda i: (i, 0)),
        out_shape=jax.ShapeDtypeStruct((E, De), jnp.float32),
    )(e, m, we_e, we_m, b_edge.reshape(1, De))

    return h_new, e_new


# speedup vs baseline: 1.9755x; 1.6504x over previous
import functools

import jax
import jax.numpy as jnp
from jax import lax
from jax.experimental import pallas as pl
from jax.experimental.pallas import tpu as pltpu
from jax.experimental.pallas import tpu_sc as plsc

_NC = 2
_NS = 16
_B = 40


def _tables_body(h_ref, wms_ref, wmr_ref, ts_ref, tr_ref):
    h = h_ref[...]
    ts_ref[...] = jnp.dot(h, wms_ref[...], preferred_element_type=jnp.float32)
    tr_ref[...] = jnp.dot(h, wmr_ref[...], preferred_element_type=jnp.float32)


def _ew_body(et_ref, wme_ref, b_ref, out_ref):
    out_ref[...] = (
        lax.dot_general(et_ref[...], wme_ref[...], (((0,), (0,)), ((), ())),
                        preferred_element_type=jnp.float32)
        + b_ref[...]
    )


def _hnew_body(h_ref, a0_ref, a1_ref, wnh_ref, wna_ref, b_ref, out_ref):
    acc = jnp.dot(h_ref[...], wnh_ref[...], preferred_element_type=jnp.float32)
    acc += jnp.dot(a0_ref[0] + a1_ref[0], wna_ref[...],
                   preferred_element_type=jnp.float32)
    out_ref[...] = jnp.maximum(acc + b_ref[...], 0.0)


def _enew_t_body(et_ref, m_ref, weet_ref, wemt_ref, bt_ref, out_ref):
    acc = jnp.dot(weet_ref[...], et_ref[...],
                  preferred_element_type=jnp.float32)
    acc += lax.dot_general(wemt_ref[...], m_ref[...], (((1,), (1,)), ((), ())),
                           preferred_element_type=jnp.float32)
    out_ref[...] = jnp.maximum(acc + bt_ref[...], 0.0)


@functools.lru_cache(maxsize=None)
def _make_sc_messages(E, N_pad, F):
    per_tile = E // (_NC * _NS)
    assert per_tile * _NC * _NS == E
    n_chunks = per_tile // _B
    assert n_chunks * _B == per_tile and n_chunks % 2 == 0
    rows_per_sub = N_pad // _NS
    assert rows_per_sub * _NS == N_pad and rows_per_sub % 8 == 0

    mesh = plsc.VectorSubcoreMesh(core_axis_name="c", subcore_axis_name="s",
                                  num_cores=_NC, num_subcores=_NS)

    @functools.partial(
        pl.kernel,
        out_type=[
            jax.ShapeDtypeStruct((E, F), jnp.float32),
            jax.ShapeDtypeStruct((_NC, N_pad, F), jnp.float32),
        ],
        mesh=mesh,
        scratch_types=[
            pltpu.VMEM((_B,), jnp.int32),
            pltpu.VMEM((_B,), jnp.int32),
            pltpu.VMEM((_B,), jnp.int32),
            pltpu.VMEM((_B,), jnp.int32),
            pltpu.VMEM((_B, F), jnp.float32),
            pltpu.VMEM((_B, F), jnp.float32),
            pltpu.VMEM((_B, F), jnp.float32),
            pltpu.VMEM((_B, F), jnp.float32),
            pltpu.VMEM((_B, F), jnp.float32),
            pltpu.VMEM((_B, F), jnp.float32),
            pltpu.VMEM_SHARED((N_pad, F), jnp.float32),
            pltpu.SemaphoreType.DMA,
            pltpu.SemaphoreType.DMA,
            pltpu.SemaphoreType.DMA,
            pltpu.SemaphoreType.DMA,
            pltpu.SemaphoreType.DMA,
            pltpu.SemaphoreType.DMA,
        ],
    )
    def sc_messages(ew, snd, rcv, ts, tr, zeros, m_out, agg_out,
                    idx_s0, idx_s1, idx_r0, idx_r1,
                    buf_ew0, buf_ew1, buf_s0, buf_s1, buf_r0, buf_r1,
                    agg_sh, isem0, isem1, gsem0, gsem1, msem0, msem1):
        idx_s = (idx_s0, idx_s1)
        idx_r = (idx_r0, idx_r1)
        buf_ew = (buf_ew0, buf_ew1)
        buf_s = (buf_s0, buf_s1)
        buf_r = (buf_r0, buf_r1)
        isem = (isem0, isem1)
        gsem = (gsem0, gsem1)
        msem = (msem0, msem1)
        cid = lax.axis_index("c")
        sid = lax.axis_index("s")
        wid = cid * _NS + sid
        base0 = wid * per_tile

        r0 = sid * rows_per_sub
        pltpu.sync_copy(zeros.at[pl.ds(r0, rows_per_sub)],
                        agg_sh.at[pl.ds(r0, rows_per_sub)])
        plsc.subcore_barrier()

        def issue_idx(c, slot):
            base = base0 + c * _B
            pltpu.async_copy(snd.at[pl.ds(base, _B)], idx_s[slot], isem[slot])
            pltpu.async_copy(rcv.at[pl.ds(base, _B)], idx_r[slot], isem[slot])

        def wait_idx(slot):
            pltpu.make_async_copy(snd.at[pl.ds(0, _B)], idx_s[slot],
                                  isem[slot]).wait()
            pltpu.make_async_copy(rcv.at[pl.ds(0, _B)], idx_r[slot],
                                  isem[slot]).wait()

        def issue_data(c, slot):
            pltpu.async_copy(ew.at[pl.ds(base0 + c * _B, _B)], buf_ew[slot],
                             gsem[slot])
            pltpu.async_copy(ts.at[idx_s[slot]], buf_s[slot], gsem[slot])
            pltpu.async_copy(tr.at[idx_r[slot]], buf_r[slot], gsem[slot])

        def wait_data(slot):
            pltpu.make_async_copy(ew.at[pl.ds(0, _B)], buf_ew[slot],
                                  gsem[slot]).wait()
            pltpu.make_async_copy(ts.at[pl.ds(0, _B)], buf_s[slot],
                                  gsem[slot]).wait()
            pltpu.make_async_copy(tr.at[pl.ds(0, _B)], buf_r[slot],
                                  gsem[slot]).wait()

        def step(g, k, o):
            @pl.when(g >= 1)
            def _():
                pltpu.make_async_copy(buf_ew[o], m_out.at[pl.ds(0, _B)],
                                      msem[o]).wait()

            @pl.when(g + 1 < n_chunks)
            def _():
                wait_idx(o)
                issue_data(g + 1, o)

            wait_data(k)

            def row_body(i, rc):
                for j in range(F // 16):
                    sl = pl.ds(j * 16, 16)
                    v = buf_ew[k][i, sl] + buf_s[k][i, sl] + buf_r[k][i, sl]
                    buf_ew[k][i, sl] = jnp.maximum(v, 0.0)
                return rc

            lax.fori_loop(0, _B, row_body, 0)
            pltpu.async_copy(buf_ew[k], m_out.at[pl.ds(base0 + g * _B, _B)],
                             msem[k])
            pltpu.sync_copy(buf_ew[k], agg_sh.at[idx_r[k]], add=True)

            @pl.when(g + 2 < n_chunks)
            def _():
                issue_idx(g + 2, k)

        issue_idx(0, 0)
        issue_idx(1, 1)
        wait_idx(0)
        issue_data(0, 0)

        def pair_body(t, carry):
            g0 = 2 * t
            step(g0, 0, 1)
            step(g0 + 1, 1, 0)
            return carry

        lax.fori_loop(0, n_chunks // 2, pair_body, 0)
        last_slot = (n_chunks - 1) % 2
        pltpu.make_async_copy(buf_ew[last_slot], m_out.at[pl.ds(0, _B)],
                              msem[last_slot]).wait()

        plsc.subcore_barrier()
        pltpu.sync_copy(agg_sh.at[pl.ds(r0, rows_per_sub)],
                        agg_out.at[cid, pl.ds(r0, rows_per_sub)])

    return sc_messages


def kernel(h, e, senders, receivers, W_msg, b_msg, W_node, b_node, W_edge, b_edge):
    N, F = h.shape
    E, De = e.shape
    senders = senders.astype(jnp.int32)
    receivers = receivers.astype(jnp.int32)

    wm_e = W_msg[:De]
    wm_s = W_msg[De:De + F]
    wm_r = W_msg[De + F:]
    wn_h = W_node[:F]
    wn_a = W_node[F:]
    we_e = W_edge[:De]
    we_m = W_edge[De:]
    b_msg2 = b_msg.reshape(1, F)
    b_node2 = b_node.reshape(1, F)

    bn = 1000
    be = 6400

    ts, tr = pl.pallas_call(
        _tables_body,
        grid=(N // bn,),
        in_specs=[
            pl.BlockSpec((bn, F), lambda i: (i, 0)),
            pl.BlockSpec((F, F), lambda i: (0, 0)),
            pl.BlockSpec((F, F), lambda i: (0, 0)),
        ],
        out_specs=[
            pl.BlockSpec((bn, F), lambda i: (i, 0)),
            pl.BlockSpec((bn, F), lambda i: (i, 0)),
        ],
        out_shape=[
            jax.ShapeDtypeStruct((N, F), jnp.float32),
            jax.ShapeDtypeStruct((N, F), jnp.float32),
        ],
    )(h, wm_s, wm_r)

    et = e.T
    ew = pl.pallas_call(
        _ew_body,
        grid=(E // be,),
        in_specs=[
            pl.BlockSpec((De, be), lambda i: (0, i)),
            pl.BlockSpec((De, F), lambda i: (0, 0)),
            pl.BlockSpec((1, F), lambda i: (0, 0)),
        ],
        out_specs=pl.BlockSpec((be, F), lambda i: (i, 0)),
        out_shape=jax.ShapeDtypeStruct((E, F), jnp.float32),
    )(et, wm_e, b_msg2)

    n_pad = ((N + (8 * _NS) - 1) // (8 * _NS)) * (8 * _NS)
    zeros = jnp.zeros((n_pad, F), jnp.float32)
    m, agg_parts = _make_sc_messages(E, n_pad, F)(
        ew, senders, receivers, ts, tr, zeros)

    h_new = pl.pallas_call(
        _hnew_body,
        grid=(N // bn,),
        in_specs=[
            pl.BlockSpec((bn, F), lambda i: (i, 0)),
            pl.BlockSpec((1, bn, F), lambda i: (0, i, 0)),
            pl.BlockSpec((1, bn, F), lambda i: (1, i, 0)),
            pl.BlockSpec((F, F), lambda i: (0, 0)),
            pl.BlockSpec((F, F), lambda i: (0, 0)),
            pl.BlockSpec((1, F), lambda i: (0, 0)),
        ],
        out_specs=pl.BlockSpec((bn, F), lambda i: (i, 0)),
        out_shape=jax.ShapeDtypeStruct((N, F), jnp.float32),
    )(h, agg_parts, agg_parts, wn_h, wn_a, b_node2)

    e_new_t = pl.pallas_call(
        _enew_t_body,
        grid=(E // be,),
        in_specs=[
            pl.BlockSpec((De, be), lambda i: (0, i)),
            pl.BlockSpec((be, F), lambda i: (i, 0)),
            pl.BlockSpec((De, De), lambda i: (0, 0)),
            pl.BlockSpec((De, F), lambda i: (0, 0)),
            pl.BlockSpec((De, 1), lambda i: (0, 0)),
        ],
        out_specs=pl.BlockSpec((De, be), lambda i: (0, i)),
        out_shape=jax.ShapeDtypeStruct((De, E), jnp.float32),
    )(et, m, we_e.T, we_m.T, b_edge.reshape(De, 1))

    return h_new, e_new_t.T


# speedup vs baseline: 2.1979x; 1.1126x over previous
import functools

import jax
import jax.numpy as jnp
from jax import lax
from jax.experimental import pallas as pl
from jax.experimental.pallas import tpu as pltpu
from jax.experimental.pallas import tpu_sc as plsc

_NC = 2
_NS = 16
_B = 40


def _tables_body(h_ref, wms_ref, wmr_ref, ts_ref, tr_ref):
    h = h_ref[...]
    ts_ref[...] = jnp.dot(h, wms_ref[...], preferred_element_type=jnp.float32)
    tr_ref[...] = jnp.dot(h, wmr_ref[...], preferred_element_type=jnp.float32)


def _ew_body(et_ref, wme_ref, b_ref, out_ref):
    out_ref[...] = (
        lax.dot_general(et_ref[...], wme_ref[...], (((0,), (0,)), ((), ())),
                        preferred_element_type=jnp.float32)
        + b_ref[...]
    )


def _hnew_body(h_ref, a0_ref, a1_ref, wnh_ref, wna_ref, b_ref, out_ref):
    acc = jnp.dot(h_ref[...], wnh_ref[...], preferred_element_type=jnp.float32)
    acc += jnp.dot(a0_ref[0] + a1_ref[0], wna_ref[...],
                   preferred_element_type=jnp.float32)
    out_ref[...] = jnp.maximum(acc + b_ref[...], 0.0)


def _enew_t_body(et_ref, m_ref, weet_ref, wemt_ref, bt_ref, out_ref):
    acc = jnp.dot(weet_ref[...], et_ref[...],
                  preferred_element_type=jnp.float32)
    acc += lax.dot_general(wemt_ref[...], m_ref[...], (((1,), (1,)), ((), ())),
                           preferred_element_type=jnp.float32)
    out_ref[...] = jnp.maximum(acc + bt_ref[...], 0.0)


@functools.lru_cache(maxsize=None)
def _make_sc_messages(E, N_pad, F):
    per_tile = E // (_NC * _NS)
    assert per_tile * _NC * _NS == E
    n_chunks = per_tile // _B
    assert n_chunks * _B == per_tile and n_chunks % 2 == 0
    rows_per_sub = N_pad // _NS
    assert rows_per_sub * _NS == N_pad and rows_per_sub % 8 == 0

    mesh = plsc.VectorSubcoreMesh(core_axis_name="c", subcore_axis_name="s",
                                  num_cores=_NC, num_subcores=_NS)

    @functools.partial(
        pl.kernel,
        out_type=[
            jax.ShapeDtypeStruct((E, F), jnp.float32),
            jax.ShapeDtypeStruct((_NC, N_pad, F), jnp.float32),
        ],
        mesh=mesh,
        scratch_types=[
            pltpu.VMEM((_B,), jnp.int32),
            pltpu.VMEM((_B,), jnp.int32),
            pltpu.VMEM((_B,), jnp.int32),
            pltpu.VMEM((_B,), jnp.int32),
            pltpu.VMEM((_B, F), jnp.float32),
            pltpu.VMEM((_B, F), jnp.float32),
            pltpu.VMEM((_B, F), jnp.float32),
            pltpu.VMEM((_B, F), jnp.float32),
            pltpu.VMEM((_B, F), jnp.float32),
            pltpu.VMEM((_B, F), jnp.float32),
            pltpu.VMEM_SHARED((N_pad, F), jnp.float32),
            pltpu.SemaphoreType.DMA,
            pltpu.SemaphoreType.DMA,
            pltpu.SemaphoreType.DMA,
            pltpu.SemaphoreType.DMA,
            pltpu.SemaphoreType.DMA,
            pltpu.SemaphoreType.DMA,
        ],
    )
    def sc_messages(ew, snd, rcv, ts, tr, zeros, m_out, agg_out,
                    idx_s0, idx_s1, idx_r0, idx_r1,
                    buf_ew0, buf_ew1, buf_s0, buf_s1, buf_r0, buf_r1,
                    agg_sh, isem0, isem1, gsem0, gsem1, msem0, msem1):
        idx_s = (idx_s0, idx_s1)
        idx_r = (idx_r0, idx_r1)
        buf_ew = (buf_ew0, buf_ew1)
        buf_s = (buf_s0, buf_s1)
        buf_r = (buf_r0, buf_r1)
        isem = (isem0, isem1)
        gsem = (gsem0, gsem1)
        msem = (msem0, msem1)
        cid = lax.axis_index("c")
        sid = lax.axis_index("s")
        wid = cid * _NS + sid
        base0 = wid * per_tile

        r0 = sid * rows_per_sub
        pltpu.sync_copy(zeros.at[pl.ds(r0, rows_per_sub)],
                        agg_sh.at[pl.ds(r0, rows_per_sub)])
        plsc.subcore_barrier()

        def issue_idx(c, slot):
            base = base0 + c * _B
            pltpu.async_copy(snd.at[pl.ds(base, _B)], idx_s[slot], isem[slot])
            pltpu.async_copy(rcv.at[pl.ds(base, _B)], idx_r[slot], isem[slot])

        def wait_idx(slot):
            pltpu.make_async_copy(snd.at[pl.ds(0, _B)], idx_s[slot],
                                  isem[slot]).wait()
            pltpu.make_async_copy(rcv.at[pl.ds(0, _B)], idx_r[slot],
                                  isem[slot]).wait()

        def issue_data(c, slot):
            pltpu.async_copy(ew.at[pl.ds(base0 + c * _B, _B)], buf_ew[slot],
                             gsem[slot])
            pltpu.async_copy(ts.at[idx_s[slot]], buf_s[slot], gsem[slot])
            pltpu.async_copy(tr.at[idx_r[slot]], buf_r[slot], gsem[slot])

        def wait_data(slot):
            pltpu.make_async_copy(ew.at[pl.ds(0, _B)], buf_ew[slot],
                                  gsem[slot]).wait()
            pltpu.make_async_copy(ts.at[pl.ds(0, _B)], buf_s[slot],
                                  gsem[slot]).wait()
            pltpu.make_async_copy(tr.at[pl.ds(0, _B)], buf_r[slot],
                                  gsem[slot]).wait()

        def step(g, k, o):
            @pl.when(g >= 1)
            def _():
                pltpu.make_async_copy(buf_ew[o], m_out.at[pl.ds(0, _B)],
                                      msem[o]).wait()

            @pl.when(g + 1 < n_chunks)
            def _():
                wait_idx(o)
                issue_data(g + 1, o)

            wait_data(k)

            @functools.partial(plsc.parallel_loop, 0, _B, unroll=2)
            def _(i):
                for j in range(F // 16):
                    sl = pl.ds(j * 16, 16)
                    v = buf_ew[k][i, sl] + buf_s[k][i, sl] + buf_r[k][i, sl]
                    buf_ew[k][i, sl] = jnp.maximum(v, 0.0)
            pltpu.async_copy(buf_ew[k], m_out.at[pl.ds(base0 + g * _B, _B)],
                             msem[k])
            pltpu.sync_copy(buf_ew[k], agg_sh.at[idx_r[k]], add=True)

            @pl.when(g + 2 < n_chunks)
            def _():
                issue_idx(g + 2, k)

        issue_idx(0, 0)
        issue_idx(1, 1)
        wait_idx(0)
        issue_data(0, 0)

        def pair_body(t, carry):
            g0 = 2 * t
            step(g0, 0, 1)
            step(g0 + 1, 1, 0)
            return carry

        lax.fori_loop(0, n_chunks // 2, pair_body, 0)
        last_slot = (n_chunks - 1) % 2
        pltpu.make_async_copy(buf_ew[last_slot], m_out.at[pl.ds(0, _B)],
                              msem[last_slot]).wait()

        plsc.subcore_barrier()
        pltpu.sync_copy(agg_sh.at[pl.ds(r0, rows_per_sub)],
                        agg_out.at[cid, pl.ds(r0, rows_per_sub)])

    return sc_messages


def kernel(h, e, senders, receivers, W_msg, b_msg, W_node, b_node, W_edge, b_edge):
    N, F = h.shape
    E, De = e.shape
    senders = senders.astype(jnp.int32)
    receivers = receivers.astype(jnp.int32)

    wm_e = W_msg[:De]
    wm_s = W_msg[De:De + F]
    wm_r = W_msg[De + F:]
    wn_h = W_node[:F]
    wn_a = W_node[F:]
    we_e = W_edge[:De]
    we_m = W_edge[De:]
    b_msg2 = b_msg.reshape(1, F)
    b_node2 = b_node.reshape(1, F)

    bn = 1000
    be = 6400

    ts, tr = pl.pallas_call(
        _tables_body,
        grid=(N // bn,),
        in_specs=[
            pl.BlockSpec((bn, F), lambda i: (i, 0)),
            pl.BlockSpec((F, F), lambda i: (0, 0)),
            pl.BlockSpec((F, F), lambda i: (0, 0)),
        ],
        out_specs=[
            pl.BlockSpec((bn, F), lambda i: (i, 0)),
            pl.BlockSpec((bn, F), lambda i: (i, 0)),
        ],
        out_shape=[
            jax.ShapeDtypeStruct((N, F), jnp.float32),
            jax.ShapeDtypeStruct((N, F), jnp.float32),
        ],
    )(h, wm_s, wm_r)

    et = e.T
    ew = pl.pallas_call(
        _ew_body,
        grid=(E // be,),
        in_specs=[
            pl.BlockSpec((De, be), lambda i: (0, i)),
            pl.BlockSpec((De, F), lambda i: (0, 0)),
            pl.BlockSpec((1, F), lambda i: (0, 0)),
        ],
        out_specs=pl.BlockSpec((be, F), lambda i: (i, 0)),
        out_shape=jax.ShapeDtypeStruct((E, F), jnp.float32),
    )(et, wm_e, b_msg2)

    n_pad = ((N + (8 * _NS) - 1) // (8 * _NS)) * (8 * _NS)
    zeros = jnp.zeros((n_pad, F), jnp.float32)
    m, agg_parts = _make_sc_messages(E, n_pad, F)(
        ew, senders, receivers, ts, tr, zeros)

    h_new = pl.pallas_call(
        _hnew_body,
        grid=(N // bn,),
        in_specs=[
            pl.BlockSpec((bn, F), lambda i: (i, 0)),
            pl.BlockSpec((1, bn, F), lambda i: (0, i, 0)),
            pl.BlockSpec((1, bn, F), lambda i: (1, i, 0)),
            pl.BlockSpec((F, F), lambda i: (0, 0)),
            pl.BlockSpec((F, F), lambda i: (0, 0)),
            pl.BlockSpec((1, F), lambda i: (0, 0)),
        ],
        out_specs=pl.BlockSpec((bn, F), lambda i: (i, 0)),
        out_shape=jax.ShapeDtypeStruct((N, F), jnp.float32),
    )(h, agg_parts, agg_parts, wn_h, wn_a, b_node2)

    e_new_t = pl.pallas_call(
        _enew_t_body,
        grid=(E // be,),
        in_specs=[
            pl.BlockSpec((De, be), lambda i: (0, i)),
            pl.BlockSpec((be, F), lambda i: (i, 0)),
            pl.BlockSpec((De, De), lambda i: (0, 0)),
            pl.BlockSpec((De, F), lambda i: (0, 0)),
            pl.BlockSpec((De, 1), lambda i: (0, 0)),
        ],
        out_specs=pl.BlockSpec((De, be), lambda i: (0, i)),
        out_shape=jax.ShapeDtypeStruct((De, E), jnp.float32),
    )(et, m, we_e.T, we_m.T, b_edge.reshape(De, 1))

    return h_new, e_new_t.T
